# baseline (speedup 1.0000x reference)
import functools

import jax
import jax.numpy as jnp
from jax.experimental import pallas as pl
from jax.experimental.pallas import tpu as pltpu

B = 64
LATENT = 32
EMB = 64
KCODES = 512


def _mm_kern(a_ref, b_ref, bias_ref, o_ref, acc_ref, *, nk, act):
    k = pl.program_id(2)

    @pl.when(k == 0)
    def _init():
        acc_ref[...] = jnp.zeros_like(acc_ref)

    acc_ref[...] += jnp.dot(a_ref[...], b_ref[...],
                            preferred_element_type=jnp.float32)

    @pl.when(k == nk - 1)
    def _fin():
        y = acc_ref[...] + bias_ref[...]
        if act == "relu":
            y = jnp.maximum(y, 0.0)
        elif act == "sigmoid":
            y = jax.nn.sigmoid(y)
        o_ref[...] = y


def _mm(a, b, bias, act, bm, bn, bk):
    M, K = a.shape
    K2, N = b.shape
    assert K == K2 and M % bm == 0 and N % bn == 0 and K % bk == 0, (a.shape, b.shape, bm, bn, bk)
    nm, nn, nk = M // bm, N // bn, K // bk
    return pl.pallas_call(
        functools.partial(_mm_kern, nk=nk, act=act),
        grid=(nm, nn, nk),
        in_specs=[
            pl.BlockSpec((bm, bk), lambda i, j, k: (i, k)),
            pl.BlockSpec((bk, bn), lambda i, j, k: (k, j)),
            pl.BlockSpec((1, bn), lambda i, j, k: (0, j)),
        ],
        out_specs=pl.BlockSpec((bm, bn), lambda i, j, k: (i, j)),
        out_shape=jax.ShapeDtypeStruct((M, N), jnp.float32),
        scratch_shapes=[pltpu.VMEM((bm, bn), jnp.float32)],
        compiler_params=pltpu.CompilerParams(
            dimension_semantics=("parallel", "parallel", "arbitrary")),
    )(a, b, bias.reshape(1, N))


def _vq_kern(z_ref, e_ref, et_ref, o_ref):
    z = z_ref[...]
    et = et_ref[...]
    scores = jnp.dot(z, et, preferred_element_type=jnp.float32)
    en2 = jnp.sum(et * et, axis=0, keepdims=True)
    dist = en2 - 2.0 * scores
    m = jnp.min(dist, axis=1, keepdims=True)
    iota = jax.lax.broadcasted_iota(jnp.int32, dist.shape, 1)
    idx = jnp.min(jnp.where(dist == m, iota, KCODES), axis=1, keepdims=True)
    onehot = (iota == idx).astype(jnp.float32)
    o_ref[...] = jnp.dot(onehot, e_ref[...],
                         preferred_element_type=jnp.float32)


def _vq(z, embeds, et, bm):
    M, _ = z.shape
    return pl.pallas_call(
        _vq_kern,
        grid=(M // bm,),
        in_specs=[
            pl.BlockSpec((bm, EMB), lambda i: (i, 0)),
            pl.BlockSpec((KCODES, EMB), lambda i: (0, 0)),
            pl.BlockSpec((EMB, KCODES), lambda i: (0, 0)),
        ],
        out_specs=pl.BlockSpec((bm, EMB), lambda i: (i, 0)),
        out_shape=jax.ShapeDtypeStruct((M, EMB), jnp.float32),
    )(z, embeds, et)


def _im2col(x):
    n, h, w, c = x.shape
    xp = jnp.pad(x, ((0, 0), (1, 1), (1, 1), (0, 0)))
    taps = [xp[:, di:di + h:2, dj:dj + w:2, :]
            for di in range(4) for dj in range(4)]
    p = jnp.concatenate(taps, axis=-1)
    return p.reshape(n * (h // 2) * (w // 2), 16 * c)


def kernel(x, Wc1, bc1, Wc2, bc2, Wc3, bc3, Wf1, bf1, Wfe, bfe, embeds,
           Wd1, bd1, Kt1, bt1, Kt2, bt2, Kt3, bt3):
    p = _im2col(x[..., None])
    h = _mm(p, Wc1.reshape(16, 64), bc1, "relu", 4096, 64, 16)
    p = _im2col(h.reshape(B, 32, 32, 64))
    h = _mm(p, Wc2.reshape(1024, 128), bc2, "relu", 1024, 128, 1024)
    p = _im2col(h.reshape(B, 16, 16, 128))
    h = _mm(p, Wc3.reshape(2048, 256), bc3, "relu", 1024, 256, 1024)
    h = _mm(h.reshape(B, 8 * 8 * 256), Wf1, bf1, "relu", B, 512, 2048)
    pe = _mm(h, Wfe, bfe, "relu", B, 1024, 1024)
    z = pe.reshape(B * LATENT, EMB)
    collected = _vq(z, embeds, embeds.T, 512)
    fc = collected.reshape(B, LATENT * EMB)
    d = _mm(fc, Wd1, bd1, "relu", B, 1024, 1024)
    w = Kt1.transpose(2, 0, 1, 3).reshape(1024, 8 * 8 * 256)
    y = _mm(d, w, jnp.tile(bt1, 64), "relu", B, 2048, 1024)
    d2 = y.reshape(B * 8 * 8, 256)
    w = Kt2.transpose(2, 0, 1, 3).reshape(256, 4 * 4 * 128)
    y = _mm(d2, w, jnp.tile(bt2, 16), "relu", 1024, 2048, 256)
    d3 = (y.reshape(B, 8, 8, 4, 4, 128)
           .transpose(0, 1, 3, 2, 4, 5)
           .reshape(B * 32 * 32, 128))
    w = jnp.zeros((128, 128), jnp.float32).at[:, :4].set(
        Kt3.transpose(2, 0, 1, 3).reshape(128, 4))
    bias = jnp.zeros((128,), jnp.float32).at[:4].set(jnp.tile(bt3, 4))
    y = _mm(d3, w, bias, "sigmoid", 4096, 128, 128)
    out = (y[:, :4].reshape(B, 32, 32, 2, 2, 1)
            .transpose(0, 1, 3, 2, 4, 5)
            .reshape(B, 64, 64, 1))
    return out


# speedup vs baseline: 17.4651x; 17.4651x over previous
import functools

import jax
import jax.numpy as jnp
from jax.experimental import pallas as pl
from jax.experimental.pallas import tpu as pltpu

B = 64
LATENT = 32
EMB = 64
KCODES = 512


def _mm_kern(a_ref, b_ref, bias_ref, o_ref, acc_ref, *, nk, act):
    k = pl.program_id(2)

    @pl.when(k == 0)
    def _init():
        acc_ref[...] = jnp.zeros_like(acc_ref)

    acc_ref[...] += jnp.dot(a_ref[...], b_ref[...],
                            preferred_element_type=jnp.float32)

    @pl.when(k == nk - 1)
    def _fin():
        y = acc_ref[...] + bias_ref[...]
        if act == "relu":
            y = jnp.maximum(y, 0.0)
        elif act == "sigmoid":
            y = jax.nn.sigmoid(y)
        o_ref[...] = y


def _mm(a, b, bias, act, bm, bn, bk):
    M, K = a.shape
    K2, N = b.shape
    assert K == K2 and M % bm == 0 and N % bn == 0 and K % bk == 0, (a.shape, b.shape, bm, bn, bk)
    nm, nn, nk = M // bm, N // bn, K // bk
    return pl.pallas_call(
        functools.partial(_mm_kern, nk=nk, act=act),
        grid=(nm, nn, nk),
        in_specs=[
            pl.BlockSpec((bm, bk), lambda i, j, k: (i, k)),
            pl.BlockSpec((bk, bn), lambda i, j, k: (k, j)),
            pl.BlockSpec((1, bn), lambda i, j, k: (0, j)),
        ],
        out_specs=pl.BlockSpec((bm, bn), lambda i, j, k: (i, j)),
        out_shape=jax.ShapeDtypeStruct((M, N), jnp.float32),
        scratch_shapes=[pltpu.VMEM((bm, bn), jnp.float32)],
        compiler_params=pltpu.CompilerParams(
            dimension_semantics=("parallel", "parallel", "arbitrary")),
    )(a, b, bias.reshape(1, N))


def _vq_kern(z_ref, e_ref, et_ref, o_ref):
    z = z_ref[...]
    et = et_ref[...]
    scores = jnp.dot(z, et, preferred_element_type=jnp.float32)
    en2 = jnp.sum(et * et, axis=0, keepdims=True)
    dist = en2 - 2.0 * scores
    m = jnp.min(dist, axis=1, keepdims=True)
    iota = jax.lax.broadcasted_iota(jnp.int32, dist.shape, 1)
    idx = jnp.min(jnp.where(dist == m, iota, KCODES), axis=1, keepdims=True)
    onehot = (iota == idx).astype(jnp.float32)
    o_ref[...] = jnp.dot(onehot, e_ref[...],
                         preferred_element_type=jnp.float32)


def _vq(z, embeds, et, bm):
    M, _ = z.shape
    return pl.pallas_call(
        _vq_kern,
        grid=(M // bm,),
        in_specs=[
            pl.BlockSpec((bm, EMB), lambda i: (i, 0)),
            pl.BlockSpec((KCODES, EMB), lambda i: (0, 0)),
            pl.BlockSpec((EMB, KCODES), lambda i: (0, 0)),
        ],
        out_specs=pl.BlockSpec((bm, EMB), lambda i: (i, 0)),
        out_shape=jax.ShapeDtypeStruct((M, EMB), jnp.float32),
    )(z, embeds, et)


def _im2col(x):
    n, h, w, c = x.shape
    hh, ww = h // 2, w // 2
    xr = x.reshape(n, hh, 2, ww, 2, c)
    ph = [[jnp.pad(xr[:, :, p, :, q, :], ((0, 0), (1, 1), (1, 1), (0, 0)))
           for q in range(2)] for p in range(2)]
    taps = []
    for di in range(4):
        a, p = divmod(di + 1, 2)
        for dj in range(4):
            b, q = divmod(dj + 1, 2)
            taps.append(ph[p][q][:, a:a + hh, b:b + ww, :])
    p_ = jnp.concatenate(taps, axis=-1)
    return p_.reshape(n * hh * ww, 16 * c)


def kernel(x, Wc1, bc1, Wc2, bc2, Wc3, bc3, Wf1, bf1, Wfe, bfe, embeds,
           Wd1, bd1, Kt1, bt1, Kt2, bt2, Kt3, bt3):
    p = _im2col(x[..., None])
    h = _mm(p, Wc1.reshape(16, 64), bc1, "relu", 4096, 64, 16)
    p = _im2col(h.reshape(B, 32, 32, 64))
    h = _mm(p, Wc2.reshape(1024, 128), bc2, "relu", 1024, 128, 1024)
    p = _im2col(h.reshape(B, 16, 16, 128))
    h = _mm(p, Wc3.reshape(2048, 256), bc3, "relu", 1024, 256, 1024)
    h = _mm(h.reshape(B, 8 * 8 * 256), Wf1, bf1, "relu", B, 512, 2048)
    pe = _mm(h, Wfe, bfe, "relu", B, 1024, 1024)
    z = pe.reshape(B * LATENT, EMB)
    collected = _vq(z, embeds, embeds.T, 512)
    fc = collected.reshape(B, LATENT * EMB)
    d = _mm(fc, Wd1, bd1, "relu", B, 1024, 1024)
    w = Kt1.transpose(2, 0, 1, 3).reshape(1024, 8 * 8 * 256)
    y = _mm(d, w, jnp.tile(bt1, 64), "relu", B, 2048, 1024)
    d2 = y.reshape(B * 8 * 8, 256)
    w = Kt2.transpose(2, 0, 1, 3).reshape(256, 4 * 4 * 128)
    y = _mm(d2, w, jnp.tile(bt2, 16), "relu", 1024, 2048, 256)
    d3 = y.reshape(B * 32 * 32, 128)
    w = jnp.zeros((128, 128), jnp.float32).at[:, :4].set(
        Kt3.transpose(2, 0, 1, 3).reshape(128, 4))
    bias = jnp.zeros((128,), jnp.float32).at[:4].set(jnp.tile(bt3, 4))
    y = _mm(d3, w, bias, "sigmoid", 4096, 128, 128)
    out = (y[:, :4].reshape(B, 8, 8, 4, 4, 2, 2)
            .transpose(0, 1, 3, 5, 2, 4, 6)
            .reshape(B, 64, 64, 1))
    return out


# speedup vs baseline: 36.3936x; 2.0838x over previous
import functools

import jax
import jax.numpy as jnp
from jax.experimental import pallas as pl
from jax.experimental.pallas import tpu as pltpu

B = 64
LATENT = 32
EMB = 64
KCODES = 512

_TAPS = [divmod(di + 1, 2) for di in range(4)]


def _c1_kern(p_ref, w_ref, b_ref, o_ref):
    o_ref[...] = jnp.maximum(
        jnp.dot(p_ref[...], w_ref[...], preferred_element_type=jnp.float32)
        + b_ref[...], 0.0)


def _conv1(p, w, bias, bm):
    M = p.shape[0]
    return pl.pallas_call(
        _c1_kern,
        grid=(M // bm,),
        in_specs=[
            pl.BlockSpec((bm, 16), lambda i: (i, 0)),
            pl.BlockSpec((16, 64), lambda i: (0, 0)),
            pl.BlockSpec((1, 64), lambda i: (0, 0)),
        ],
        out_specs=pl.BlockSpec((bm, 64), lambda i: (i, 0)),
        out_shape=jax.ShapeDtypeStruct((M, 64), jnp.float32),
    )(p, w, bias.reshape(1, 64))


def _cphase_kern(x_ref, w_ref, b_ref, o_ref, *, nb, hh, cin, cout):
    rows = nb * hh * hh
    acc = jnp.zeros((rows, cout), jnp.float32)
    for di in range(4):
        a, p = _TAPS[di]
        for dj in range(4):
            b_, q = _TAPS[dj]
            tap = x_ref[p, q, :, a:a + hh, b_:b_ + hh, :].reshape(rows, cin)
            acc += jnp.dot(tap, w_ref[di * 4 + dj],
                           preferred_element_type=jnp.float32)
    o_ref[...] = jnp.maximum(acc + b_ref[...], 0.0)


def _conv_phase(xph, w, bias, nb, hh, cin, cout):
    ng = B // nb
    kern = functools.partial(_cphase_kern, nb=nb, hh=hh, cin=cin, cout=cout)
    return pl.pallas_call(
        kern,
        grid=(ng,),
        in_specs=[
            pl.BlockSpec((2, 2, nb, hh + 2, hh + 2, cin),
                         lambda g: (0, 0, g, 0, 0, 0)),
            pl.BlockSpec((16, cin, cout), lambda g: (0, 0, 0)),
            pl.BlockSpec((1, cout), lambda g: (0, 0)),
        ],
        out_specs=pl.BlockSpec((nb * hh * hh, cout), lambda g: (g, 0)),
        out_shape=jax.ShapeDtypeStruct((B * hh * hh, cout), jnp.float32),
    )(xph, w, bias.reshape(1, cout))


_S_MID = 16
_S_T1 = 17
_S_T23 = 49
_NSTEP = 57


def _mega_kern(h3_ref, wf1_ref, bf1_ref, wfe_ref, bfe_ref, e_ref, et_ref,
               wd1_ref, bd1_ref, kt1_ref, bt1_ref, kt2_ref, bt2_ref,
               bd3_ref, bt3_ref, o_ref, acc_ref, d_ref, t1_ref):
    s = pl.program_id(0)

    @pl.when(s == 0)
    def _init():
        acc_ref[...] = jnp.zeros_like(acc_ref)

    @pl.when(s < _S_MID)
    def _fc1():
        acc_ref[...] += jnp.dot(h3_ref[...], wf1_ref[...],
                                preferred_element_type=jnp.float32)

    @pl.when(s == _S_MID)
    def _mid():
        h1 = jnp.maximum(acc_ref[...] + bf1_ref[...], 0.0)
        pe = jnp.maximum(
            jnp.dot(h1, wfe_ref[...], preferred_element_type=jnp.float32)
            + bfe_ref[...], 0.0)
        et = et_ref[...]
        e = e_ref[...]
        en2 = jnp.sum(et * et, axis=0, keepdims=True)
        cols = []
        for latent in range(LATENT):
            zl = pe[:, latent * EMB:(latent + 1) * EMB]
            dist = en2 - 2.0 * jnp.dot(zl, et,
                                       preferred_element_type=jnp.float32)
            mn = jnp.min(dist, axis=1, keepdims=True)
            iota = jax.lax.broadcasted_iota(jnp.int32, dist.shape, 1)
            idx = jnp.min(jnp.where(dist == mn, iota, KCODES), axis=1,
                          keepdims=True)
            onehot = (iota == idx).astype(jnp.float32)
            cols.append(jnp.dot(onehot, e,
                                preferred_element_type=jnp.float32))
        fcv = jnp.concatenate(cols, axis=1)
        d_ref[...] = jnp.maximum(
            jnp.dot(fcv, wd1_ref[...], preferred_element_type=jnp.float32)
            + bd1_ref[...], 0.0)

    @pl.when((s >= _S_T1) & (s < _S_T23))
    def _t1():
        j = s - _S_T1
        d = d_ref[...]
        kt = kt1_ref[...]
        y0 = jnp.dot(d, kt[0], preferred_element_type=jnp.float32)
        y1 = jnp.dot(d, kt[1], preferred_element_type=jnp.float32)
        t1_ref[pl.ds(j * 128, 64), :] = jnp.maximum(y0 + bt1_ref[...], 0.0)
        t1_ref[pl.ds(j * 128 + 64, 64), :] = jnp.maximum(y1 + bt1_ref[...], 0.0)

    @pl.when(s >= _S_T23)
    def _t23():
        mred = s - _S_T23
        a2 = t1_ref[pl.ds(mred * 512, 512), :]
        y2 = jnp.maximum(
            jnp.dot(a2, kt2_ref[...], preferred_element_type=jnp.float32)
            + bt2_ref[...], 0.0)
        o_ref[...] = jax.nn.sigmoid(
            jnp.dot(y2, bd3_ref[...], preferred_element_type=jnp.float32)
            + bt3_ref[...])


def _mega(h3, Wf1, bf1, Wfe, bfe, embeds, et, Wd1, bd1, kt1r, bt1r,
          kt2r, bt2r, bd3, bt3r):
    c = lambda s: (0, 0)
    c3 = lambda s: (0, 0, 0)
    return pl.pallas_call(
        _mega_kern,
        grid=(_NSTEP,),
        in_specs=[
            pl.BlockSpec((B, 1024), lambda s: (0, jnp.minimum(s, 15))),
            pl.BlockSpec((1024, 1024), lambda s: (jnp.minimum(s, 15), 0)),
            pl.BlockSpec((1, 1024), c),
            pl.BlockSpec((1024, 2048), c),
            pl.BlockSpec((1, 2048), c),
            pl.BlockSpec((KCODES, EMB), c),
            pl.BlockSpec((EMB, KCODES), c),
            pl.BlockSpec((2048, 1024), c),
            pl.BlockSpec((1, 1024), c),
            pl.BlockSpec((2, 1024, 256),
                         lambda s: (jnp.clip(s - _S_T1, 0, 31), 0, 0)),
            pl.BlockSpec((1, 256), c),
            pl.BlockSpec((256, 2048), c),
            pl.BlockSpec((1, 2048), c),
            pl.BlockSpec((2048, 64), c),
            pl.BlockSpec((1, 64), c),
        ],
        out_specs=pl.BlockSpec((512, 64),
                               lambda s: (jnp.clip(s - _S_T23, 0, 7), 0)),
        out_shape=jax.ShapeDtypeStruct((B * 64, 64), jnp.float32),
        scratch_shapes=[
            pltpu.VMEM((B, 1024), jnp.float32),
            pltpu.VMEM((B, 1024), jnp.float32),
            pltpu.VMEM((4096, 256), jnp.float32),
        ],
        compiler_params=pltpu.CompilerParams(
            dimension_semantics=("arbitrary",)),
    )(h3, Wf1, bf1.reshape(1, 1024), Wfe, bfe.reshape(1, 2048), embeds, et,
      Wd1, bd1.reshape(1, 1024), kt1r, bt1r, kt2r, bt2r, bd3, bt3r)


def _final_transpose(y):
    return (y.reshape(8, 8, B, 4, 4, 2, 2)
             .transpose(2, 0, 3, 5, 1, 4, 6)
             .reshape(B, 64, 64, 1))


def _im2col16(x):
    hh = 32
    xr = x.reshape(B, hh, 2, hh, 2)
    ph = [[jnp.pad(xr[:, :, p, :, q], ((0, 0), (1, 1), (1, 1)))
           for q in range(2)] for p in range(2)]
    taps = []
    for di in range(4):
        a, p = _TAPS[di]
        for dj in range(4):
            b_, q = _TAPS[dj]
            taps.append(ph[p][q][:, a:a + hh, b_:b_ + hh, None])
    return jnp.concatenate(taps, axis=-1).reshape(B * hh * hh, 16)


def _phase_pad(h, hh, c):
    t = (h.reshape(B, hh, 2, hh, 2, c)
          .transpose(2, 4, 0, 1, 3, 5))
    return jnp.pad(t, ((0, 0), (0, 0), (0, 0), (1, 1), (1, 1), (0, 0)))


def kernel(x, Wc1, bc1, Wc2, bc2, Wc3, bc3, Wf1, bf1, Wfe, bfe, embeds,
           Wd1, bd1, Kt1, bt1, Kt2, bt2, Kt3, bt3):
    h = _conv1(_im2col16(x), Wc1.reshape(16, 64), bc1, 4096)
    h = _conv_phase(_phase_pad(h, 16, 64), Wc2.reshape(16, 64, 128),
                    bc2, 8, 16, 64, 128)
    h = _conv_phase(_phase_pad(h, 8, 128), Wc3.reshape(16, 128, 256),
                    bc3, 16, 8, 128, 256)
    kt1r = Kt1.reshape(64, 1024, 256)
    bt1r = bt1.reshape(1, 256)
    kt2r = Kt2.transpose(2, 0, 1, 3).reshape(256, 2048)
    bt2r = jnp.tile(bt2, 16).reshape(1, 2048)
    bd3 = jnp.kron(jnp.eye(16, dtype=jnp.float32),
                   Kt3.transpose(2, 0, 1, 3).reshape(128, 4))
    bt3r = jnp.tile(bt3, 64).reshape(1, 64)
    y = _mega(h.reshape(B, 16384), Wf1, bf1, Wfe, bfe, embeds, embeds.T,
              Wd1, bd1, kt1r, bt1r, kt2r, bt2r, bd3, bt3r)
    return _final_transpose(y)


# speedup vs baseline: 38.5444x; 1.0591x over previous
import functools

import jax
import jax.numpy as jnp
from jax.experimental import pallas as pl
from jax.experimental.pallas import tpu as pltpu

B = 64
LATENT = 32
EMB = 64
KCODES = 512

_TAPS = [divmod(di + 1, 2) for di in range(4)]


def _c1_kern(p_ref, w_ref, b_ref, o_ref):
    o_ref[...] = jnp.maximum(
        jnp.dot(p_ref[...], w_ref[...], preferred_element_type=jnp.float32)
        + b_ref[...], 0.0)


def _conv1(p, w, bias, bm):
    M = p.shape[0]
    return pl.pallas_call(
        _c1_kern,
        grid=(M // bm,),
        in_specs=[
            pl.BlockSpec((bm, 16), lambda i: (i, 0)),
            pl.BlockSpec((16, 64), lambda i: (0, 0)),
            pl.BlockSpec((1, 64), lambda i: (0, 0)),
        ],
        out_specs=pl.BlockSpec((bm, 64), lambda i: (i, 0)),
        out_shape=jax.ShapeDtypeStruct((M, 64), jnp.float32),
    )(p, w, bias.reshape(1, 64))


def _cphase_kern(x_ref, w_ref, b_ref, o_ref, *, nb, hh, cin, cout):
    rows = nb * hh * hh
    acc = jnp.zeros((rows, cout), jnp.float32)
    for di in range(4):
        a, p = _TAPS[di]
        for dj in range(4):
            b_, q = _TAPS[dj]
            tap = x_ref[p, q, :, a:a + hh, b_:b_ + hh, :].reshape(rows, cin)
            acc += jnp.dot(tap, w_ref[di * 4 + dj],
                           preferred_element_type=jnp.float32)
    o_ref[...] = jnp.maximum(acc + b_ref[...], 0.0)


def _c23_kern(x_ref, w2_ref, b2_ref, w3_ref, b3_ref, o_ref, c2_ref, *, nb):
    rows = nb * 64

    @pl.when(pl.program_id(0) == 0)
    def _zero():
        c2_ref[...] = jnp.zeros_like(c2_ref)

    for P in range(2):
        for Q in range(2):
            acc = jnp.zeros((rows, 128), jnp.float32)
            for di in range(4):
                t = 2 * P + di - 1
                ri, oi = t % 4, (t - t % 4) // 4 + 1
                for dj in range(4):
                    u = 2 * Q + dj - 1
                    rj, oj = u % 4, (u - u % 4) // 4 + 1
                    tap = x_ref[ri, rj, :, oi:oi + 8, oj:oj + 8, :]
                    acc += jnp.dot(tap.reshape(rows, 64), w2_ref[di * 4 + dj],
                                   preferred_element_type=jnp.float32)
            y = jnp.maximum(acc + b2_ref[...], 0.0)
            c2_ref[P, Q, :, 1:9, 1:9, :] = y.reshape(nb, 8, 8, 128)
    acc3 = jnp.zeros((rows, 256), jnp.float32)
    for di in range(4):
        a, p = _TAPS[di]
        for dj in range(4):
            b_, q = _TAPS[dj]
            tap = c2_ref[p, q, :, a:a + 8, b_:b_ + 8, :]
            acc3 += jnp.dot(tap.reshape(rows, 128), w3_ref[di * 4 + dj],
                            preferred_element_type=jnp.float32)
    o_ref[...] = jnp.maximum(acc3 + b3_ref[...], 0.0)


def _conv23(xph4, w2, b2, w3, b3, nb):
    ng = B // nb
    return pl.pallas_call(
        functools.partial(_c23_kern, nb=nb),
        grid=(ng,),
        in_specs=[
            pl.BlockSpec((4, 4, nb, 10, 10, 64),
                         lambda g: (0, 0, g, 0, 0, 0)),
            pl.BlockSpec((16, 64, 128), lambda g: (0, 0, 0)),
            pl.BlockSpec((1, 128), lambda g: (0, 0)),
            pl.BlockSpec((16, 128, 256), lambda g: (0, 0, 0)),
            pl.BlockSpec((1, 256), lambda g: (0, 0)),
        ],
        out_specs=pl.BlockSpec((nb * 64, 256), lambda g: (g, 0)),
        out_shape=jax.ShapeDtypeStruct((B * 64, 256), jnp.float32),
        scratch_shapes=[pltpu.VMEM((2, 2, nb, 10, 10, 128), jnp.float32)],
        compiler_params=pltpu.CompilerParams(
            dimension_semantics=("arbitrary",)),
    )(xph4, w2, b2.reshape(1, 128), w3, b3.reshape(1, 256))


def _conv_phase(xph, w, bias, nb, hh, cin, cout):
    ng = B // nb
    kern = functools.partial(_cphase_kern, nb=nb, hh=hh, cin=cin, cout=cout)
    return pl.pallas_call(
        kern,
        grid=(ng,),
        in_specs=[
            pl.BlockSpec((2, 2, nb, hh + 2, hh + 2, cin),
                         lambda g: (0, 0, g, 0, 0, 0)),
            pl.BlockSpec((16, cin, cout), lambda g: (0, 0, 0)),
            pl.BlockSpec((1, cout), lambda g: (0, 0)),
        ],
        out_specs=pl.BlockSpec((nb * hh * hh, cout), lambda g: (g, 0)),
        out_shape=jax.ShapeDtypeStruct((B * hh * hh, cout), jnp.float32),
    )(xph, w, bias.reshape(1, cout))


_S_MID = 16
_S_T1 = 17
_S_T23 = 49
_NSTEP = 57


def _mega_kern(h3_ref, wf1_ref, bf1_ref, wfe_ref, bfe_ref, e_ref, et_ref,
               wd1_ref, bd1_ref, kt1_ref, bt1_ref, kt2_ref, bt2_ref,
               bd3_ref, bt3_ref, o_ref, acc_ref, d_ref, t1_ref):
    s = pl.program_id(0)

    @pl.when(s == 0)
    def _init():
        acc_ref[...] = jnp.zeros_like(acc_ref)

    @pl.when(s < _S_MID)
    def _fc1():
        acc_ref[...] += jnp.dot(h3_ref[...], wf1_ref[...],
                                preferred_element_type=jnp.float32)

    @pl.when(s == _S_MID)
    def _mid():
        h1 = jnp.maximum(acc_ref[...] + bf1_ref[...], 0.0)
        pe = jnp.maximum(
            jnp.dot(h1, wfe_ref[...], preferred_element_type=jnp.float32)
            + bfe_ref[...], 0.0)
        et = et_ref[...]
        e = e_ref[...]
        en2 = jnp.sum(et * et, axis=0, keepdims=True)
        cols = []
        for latent in range(LATENT):
            zl = pe[:, latent * EMB:(latent + 1) * EMB]
            dist = en2 - 2.0 * jnp.dot(zl, et,
                                       preferred_element_type=jnp.float32)
            mn = jnp.min(dist, axis=1, keepdims=True)
            iota = jax.lax.broadcasted_iota(jnp.int32, dist.shape, 1)
            idx = jnp.min(jnp.where(dist == mn, iota, KCODES), axis=1,
                          keepdims=True)
            onehot = (iota == idx).astype(jnp.float32)
            cols.append(jnp.dot(onehot, e,
                                preferred_element_type=jnp.float32))
        fcv = jnp.concatenate(cols, axis=1)
        d_ref[...] = jnp.maximum(
            jnp.dot(fcv, wd1_ref[...], preferred_element_type=jnp.float32)
            + bd1_ref[...], 0.0)

    @pl.when((s >= _S_T1) & (s < _S_T23))
    def _t1():
        j = s - _S_T1
        d = d_ref[...]
        kt = kt1_ref[...]
        y0 = jnp.dot(d, kt[0], preferred_element_type=jnp.float32)
        y1 = jnp.dot(d, kt[1], preferred_element_type=jnp.float32)
        t1_ref[pl.ds(j * 128, 64), :] = jnp.maximum(y0 + bt1_ref[...], 0.0)
        t1_ref[pl.ds(j * 128 + 64, 64), :] = jnp.maximum(y1 + bt1_ref[...], 0.0)

    @pl.when(s >= _S_T23)
    def _t23():
        mred = s - _S_T23
        a2 = t1_ref[pl.ds(mred * 512, 512), :]
        y2 = jnp.maximum(
            jnp.dot(a2, kt2_ref[...], preferred_element_type=jnp.float32)
            + bt2_ref[...], 0.0)
        o_ref[...] = jax.nn.sigmoid(
            jnp.dot(y2, bd3_ref[...], preferred_element_type=jnp.float32)
            + bt3_ref[...])


def _mega(h3, Wf1, bf1, Wfe, bfe, embeds, et, Wd1, bd1, kt1r, bt1r,
          kt2r, bt2r, bd3, bt3r):
    c = lambda s: (0, 0)
    c3 = lambda s: (0, 0, 0)
    return pl.pallas_call(
        _mega_kern,
        grid=(_NSTEP,),
        in_specs=[
            pl.BlockSpec((B, 1024), lambda s: (0, jnp.minimum(s, 15))),
            pl.BlockSpec((1024, 1024), lambda s: (jnp.minimum(s, 15), 0)),
            pl.BlockSpec((1, 1024), c),
            pl.BlockSpec((1024, 2048), c),
            pl.BlockSpec((1, 2048), c),
            pl.BlockSpec((KCODES, EMB), c),
            pl.BlockSpec((EMB, KCODES), c),
            pl.BlockSpec((2048, 1024), c),
            pl.BlockSpec((1, 1024), c),
            pl.BlockSpec((2, 1024, 256),
                         lambda s: (jnp.clip(s - _S_T1, 0, 31), 0, 0)),
            pl.BlockSpec((1, 256), c),
            pl.BlockSpec((256, 2048), c),
            pl.BlockSpec((1, 2048), c),
            pl.BlockSpec((2048, 64), c),
            pl.BlockSpec((1, 64), c),
        ],
        out_specs=pl.BlockSpec((512, 64),
                               lambda s: (jnp.clip(s - _S_T23, 0, 7), 0)),
        out_shape=jax.ShapeDtypeStruct((B * 64, 64), jnp.float32),
        scratch_shapes=[
            pltpu.VMEM((B, 1024), jnp.float32),
            pltpu.VMEM((B, 1024), jnp.float32),
            pltpu.VMEM((4096, 256), jnp.float32),
        ],
        compiler_params=pltpu.CompilerParams(
            dimension_semantics=("arbitrary",)),
    )(h3, Wf1, bf1.reshape(1, 1024), Wfe, bfe.reshape(1, 2048), embeds, et,
      Wd1, bd1.reshape(1, 1024), kt1r, bt1r, kt2r, bt2r, bd3, bt3r)


def _final_transpose(y):
    return (y.reshape(8, 8, B, 4, 4, 2, 2)
             .transpose(2, 0, 3, 5, 1, 4, 6)
             .reshape(B, 64, 64, 1))


def _im2col16(x):
    hh = 32
    xr = x.reshape(B, hh, 2, hh, 2)
    ph = [[jnp.pad(xr[:, :, p, :, q], ((0, 0), (1, 1), (1, 1)))
           for q in range(2)] for p in range(2)]
    taps = []
    for di in range(4):
        a, p = _TAPS[di]
        for dj in range(4):
            b_, q = _TAPS[dj]
            taps.append(ph[p][q][:, a:a + hh, b_:b_ + hh, None])
    return jnp.concatenate(taps, axis=-1).reshape(B * hh * hh, 16)


def _phase_pad(h, hh, c):
    t = (h.reshape(B, hh, 2, hh, 2, c)
          .transpose(2, 4, 0, 1, 3, 5))
    return jnp.pad(t, ((0, 0), (0, 0), (0, 0), (1, 1), (1, 1), (0, 0)))


def kernel(x, Wc1, bc1, Wc2, bc2, Wc3, bc3, Wf1, bf1, Wfe, bfe, embeds,
           Wd1, bd1, Kt1, bt1, Kt2, bt2, Kt3, bt3):
    h = _conv1(_im2col16(x), Wc1.reshape(16, 64), bc1, 4096)
    xph4 = jnp.pad(h.reshape(B, 8, 4, 8, 4, 64).transpose(2, 4, 0, 1, 3, 5),
                   ((0, 0), (0, 0), (0, 0), (1, 1), (1, 1), (0, 0)))
    h = _conv23(xph4, Wc2.reshape(16, 64, 128), bc2,
                Wc3.reshape(16, 128, 256), bc3, 16)
    kt1r = Kt1.reshape(64, 1024, 256)
    bt1r = bt1.reshape(1, 256)
    kt2r = Kt2.transpose(2, 0, 1, 3).reshape(256, 2048)
    bt2r = jnp.tile(bt2, 16).reshape(1, 2048)
    bd3 = jnp.kron(jnp.eye(16, dtype=jnp.float32),
                   Kt3.transpose(2, 0, 1, 3).reshape(128, 4))
    bt3r = jnp.tile(bt3, 64).reshape(1, 64)
    y = _mega(h.reshape(B, 16384), Wf1, bf1, Wfe, bfe, embeds, embeds.T,
              Wd1, bd1, kt1r, bt1r, kt2r, bt2r, bd3, bt3r)
    return _final_transpose(y)


# speedup vs baseline: 65.0196x; 1.6869x over previous
import functools

import jax
import jax.numpy as jnp
from jax.experimental import pallas as pl
from jax.experimental.pallas import tpu as pltpu

B = 64
LATENT = 32
EMB = 64
KCODES = 512

_TAPS = [divmod(di + 1, 2) for di in range(4)]


def _enc_kern(xp_ref, w1_ref, b1_ref, w2_ref, b2_ref, w3_ref, b3_ref,
              o_ref, c1_ref, c2_ref, *, nb):
    rows = nb * 64

    @pl.when(pl.program_id(0) == 0)
    def _zero():
        c1_ref[...] = jnp.zeros_like(c1_ref)
        c2_ref[...] = jnp.zeros_like(c2_ref)

    xpr = xp_ref[...].reshape(nb, 4, 4, 8, 8, 16)
    for ri in range(4):
        for rj in range(4):
            y = jnp.maximum(
                jnp.dot(xpr[:, ri, rj].reshape(rows, 16), w1_ref[...],
                        preferred_element_type=jnp.float32) + b1_ref[...],
                0.0)
            c1_ref[ri, rj, :, 1:9, 1:9, :] = y.reshape(nb, 8, 8, 64)
    for P in range(2):
        for Q in range(2):
            acc = jnp.zeros((rows, 128), jnp.float32)
            for di in range(4):
                t = 2 * P + di - 1
                ri, oi = t % 4, (t - t % 4) // 4 + 1
                for dj in range(4):
                    u = 2 * Q + dj - 1
                    rj, oj = u % 4, (u - u % 4) // 4 + 1
                    tap = c1_ref[ri, rj, :, oi:oi + 8, oj:oj + 8, :]
                    acc += jnp.dot(tap.reshape(rows, 64), w2_ref[di * 4 + dj],
                                   preferred_element_type=jnp.float32)
            y = jnp.maximum(acc + b2_ref[...], 0.0)
            c2_ref[P, Q, :, 1:9, 1:9, :] = y.reshape(nb, 8, 8, 128)
    acc3 = jnp.zeros((rows, 256), jnp.float32)
    for di in range(4):
        a, p = _TAPS[di]
        for dj in range(4):
            b_, q = _TAPS[dj]
            tap = c2_ref[p, q, :, a:a + 8, b_:b_ + 8, :]
            acc3 += jnp.dot(tap.reshape(rows, 128), w3_ref[di * 4 + dj],
                            preferred_element_type=jnp.float32)
    o_ref[...] = jnp.maximum(acc3 + b3_ref[...], 0.0)


def _encoder(xp, w1, b1, w2, b2, w3, b3, nb):
    ng = B // nb
    return pl.pallas_call(
        functools.partial(_enc_kern, nb=nb),
        grid=(ng,),
        in_specs=[
            pl.BlockSpec((nb * 1024, 16), lambda g: (g, 0)),
            pl.BlockSpec((16, 64), lambda g: (0, 0)),
            pl.BlockSpec((1, 64), lambda g: (0, 0)),
            pl.BlockSpec((16, 64, 128), lambda g: (0, 0, 0)),
            pl.BlockSpec((1, 128), lambda g: (0, 0)),
            pl.BlockSpec((16, 128, 256), lambda g: (0, 0, 0)),
            pl.BlockSpec((1, 256), lambda g: (0, 0)),
        ],
        out_specs=pl.BlockSpec((nb * 64, 256), lambda g: (g, 0)),
        out_shape=jax.ShapeDtypeStruct((B * 64, 256), jnp.float32),
        scratch_shapes=[
            pltpu.VMEM((4, 4, nb, 10, 10, 64), jnp.float32),
            pltpu.VMEM((2, 2, nb, 10, 10, 128), jnp.float32),
        ],
        compiler_params=pltpu.CompilerParams(
            dimension_semantics=("arbitrary",)),
    )(xp, w1, b1.reshape(1, 64), w2, b2.reshape(1, 128),
      w3, b3.reshape(1, 256))


_S_MID = 16
_S_T1 = 17
_S_T23 = 49
_NSTEP = 57


def _mega_kern(h3_ref, wf1_ref, bf1_ref, wfe_ref, bfe_ref, e_ref, et_ref,
               wd1_ref, bd1_ref, kt1_ref, bt1_ref, kt2_ref, bt2_ref,
               bd3_ref, bt3_ref, o_ref, acc_ref, d_ref, t1_ref):
    s = pl.program_id(0)

    @pl.when(s == 0)
    def _init():
        acc_ref[...] = jnp.zeros_like(acc_ref)

    @pl.when(s < _S_MID)
    def _fc1():
        acc_ref[...] += jnp.dot(h3_ref[...], wf1_ref[...],
                                preferred_element_type=jnp.float32)

    @pl.when(s == _S_MID)
    def _mid():
        h1 = jnp.maximum(acc_ref[...] + bf1_ref[...], 0.0)
        pe = jnp.maximum(
            jnp.dot(h1, wfe_ref[...], preferred_element_type=jnp.float32)
            + bfe_ref[...], 0.0)
        et = et_ref[...]
        e = e_ref[...]
        en2 = jnp.sum(et * et, axis=0, keepdims=True)
        cols = []
        for latent in range(LATENT):
            zl = pe[:, latent * EMB:(latent + 1) * EMB]
            dist = en2 - 2.0 * jnp.dot(zl, et,
                                       preferred_element_type=jnp.float32)
            mn = jnp.min(dist, axis=1, keepdims=True)
            iota = jax.lax.broadcasted_iota(jnp.int32, dist.shape, 1)
            idx = jnp.min(jnp.where(dist == mn, iota, KCODES), axis=1,
                          keepdims=True)
            onehot = (iota == idx).astype(jnp.float32)
            cols.append(jnp.dot(onehot, e,
                                preferred_element_type=jnp.float32))
        fcv = jnp.concatenate(cols, axis=1)
        d_ref[...] = jnp.maximum(
            jnp.dot(fcv, wd1_ref[...], preferred_element_type=jnp.float32)
            + bd1_ref[...], 0.0)

    @pl.when((s >= _S_T1) & (s < _S_T23))
    def _t1():
        j = s - _S_T1
        d = d_ref[...]
        kt = kt1_ref[...]
        y0 = jnp.dot(d, kt[0], preferred_element_type=jnp.float32)
        y1 = jnp.dot(d, kt[1], preferred_element_type=jnp.float32)
        t1_ref[pl.ds(j * 128, 64), :] = jnp.maximum(y0 + bt1_ref[...], 0.0)
        t1_ref[pl.ds(j * 128 + 64, 64), :] = jnp.maximum(y1 + bt1_ref[...], 0.0)

    @pl.when(s >= _S_T23)
    def _t23():
        mred = s - _S_T23
        a2 = t1_ref[pl.ds(mred * 512, 512), :]
        y2 = jnp.maximum(
            jnp.dot(a2, kt2_ref[...], preferred_element_type=jnp.float32)
            + bt2_ref[...], 0.0)
        o_ref[...] = jax.nn.sigmoid(
            jnp.dot(y2, bd3_ref[...], preferred_element_type=jnp.float32)
            + bt3_ref[...])


def _mega(h3, Wf1, bf1, Wfe, bfe, embeds, et, Wd1, bd1, kt1r, bt1r,
          kt2r, bt2r, bd3, bt3r):
    c = lambda s: (0, 0)
    c3 = lambda s: (0, 0, 0)
    return pl.pallas_call(
        _mega_kern,
        grid=(_NSTEP,),
        in_specs=[
            pl.BlockSpec((B, 1024), lambda s: (0, jnp.minimum(s, 15))),
            pl.BlockSpec((1024, 1024), lambda s: (jnp.minimum(s, 15), 0)),
            pl.BlockSpec((1, 1024), c),
            pl.BlockSpec((1024, 2048), c),
            pl.BlockSpec((1, 2048), c),
            pl.BlockSpec((KCODES, EMB), c),
            pl.BlockSpec((EMB, KCODES), c),
            pl.BlockSpec((2048, 1024), c),
            pl.BlockSpec((1, 1024), c),
            pl.BlockSpec((2, 1024, 256),
                         lambda s: (jnp.clip(s - _S_T1, 0, 31), 0, 0)),
            pl.BlockSpec((1, 256), c),
            pl.BlockSpec((256, 2048), c),
            pl.BlockSpec((1, 2048), c),
            pl.BlockSpec((2048, 64), c),
            pl.BlockSpec((1, 64), c),
        ],
        out_specs=pl.BlockSpec((512, 64),
                               lambda s: (jnp.clip(s - _S_T23, 0, 7), 0)),
        out_shape=jax.ShapeDtypeStruct((B * 64, 64), jnp.float32),
        scratch_shapes=[
            pltpu.VMEM((B, 1024), jnp.float32),
            pltpu.VMEM((B, 1024), jnp.float32),
            pltpu.VMEM((4096, 256), jnp.float32),
        ],
        compiler_params=pltpu.CompilerParams(
            dimension_semantics=("arbitrary",)),
    )(h3, Wf1, bf1.reshape(1, 1024), Wfe, bfe.reshape(1, 2048), embeds, et,
      Wd1, bd1.reshape(1, 1024), kt1r, bt1r, kt2r, bt2r, bd3, bt3r)


def _final_transpose(y):
    return (y.reshape(8, 8, B, 4, 4, 2, 2)
             .transpose(2, 0, 3, 5, 1, 4, 6)
             .reshape(B, 64, 64, 1))


def _im2col8(x):
    x8p = jnp.pad(x.reshape(B, 8, 8, 8, 8).transpose(2, 4, 0, 1, 3),
                  ((0, 0), (0, 0), (0, 0), (1, 1), (1, 1)))
    def off(r, d):
        t = 2 * r + d - 1
        s = t % 8
        return s, (t - s) // 8 + 1
    ris = []
    for ri in range(4):
        rjs = []
        for rj in range(4):
            taps = []
            for di in range(4):
                si, oi = off(ri, di)
                for dj in range(4):
                    sj, oj = off(rj, dj)
                    taps.append(x8p[si, sj, :, oi:oi + 8, oj:oj + 8])
            rjs.append(jnp.stack(taps, axis=-1))
        ris.append(jnp.stack(rjs, axis=1))
    return jnp.stack(ris, axis=1).reshape(B * 1024, 16)


def kernel(x, Wc1, bc1, Wc2, bc2, Wc3, bc3, Wf1, bf1, Wfe, bfe, embeds,
           Wd1, bd1, Kt1, bt1, Kt2, bt2, Kt3, bt3):
    h = _encoder(_im2col8(x), Wc1.reshape(16, 64), bc1,
                 Wc2.reshape(16, 64, 128), bc2,
                 Wc3.reshape(16, 128, 256), bc3, 16)
    kt1r = Kt1.reshape(64, 1024, 256)
    bt1r = bt1.reshape(1, 256)
    kt2r = Kt2.transpose(2, 0, 1, 3).reshape(256, 2048)
    bt2r = jnp.tile(bt2, 16).reshape(1, 2048)
    bd3 = jnp.kron(jnp.eye(16, dtype=jnp.float32),
                   Kt3.transpose(2, 0, 1, 3).reshape(128, 4))
    bt3r = jnp.tile(bt3, 64).reshape(1, 64)
    y = _mega(h.reshape(B, 16384), Wf1, bf1, Wfe, bfe, embeds, embeds.T,
              Wd1, bd1, kt1r, bt1r, kt2r, bt2r, bd3, bt3r)
    return _final_transpose(y)


# speedup vs baseline: 68.5632x; 1.0545x over previous
import functools

import jax
import jax.numpy as jnp
from jax.experimental import pallas as pl
from jax.experimental.pallas import tpu as pltpu

B = 64
LATENT = 32
EMB = 64
KCODES = 512

_TAPS = [divmod(di + 1, 2) for di in range(4)]


def _enc_kern(xp_ref, w1_ref, b1_ref, w2_ref, b2_ref, w3_ref, b3_ref,
              o_ref, c1_ref, c2_ref, *, nb):
    rows = nb * 64

    @pl.when(pl.program_id(0) == 0)
    def _zero():
        c1_ref[...] = jnp.zeros_like(c1_ref)
        c2_ref[...] = jnp.zeros_like(c2_ref)

    xpr = xp_ref[...].reshape(nb, 4, 4, 8, 8, 16)
    for ri in range(4):
        for rj in range(4):
            y = jnp.maximum(
                jnp.dot(xpr[:, ri, rj].reshape(rows, 16), w1_ref[...],
                        preferred_element_type=jnp.float32) + b1_ref[...],
                0.0)
            c1_ref[ri, rj, :, 1:9, 1:9, :] = y.reshape(nb, 8, 8, 64)
    for P in range(2):
        for Q in range(2):
            acc = jnp.zeros((rows, 128), jnp.float32)
            for di in range(4):
                t = 2 * P + di - 1
                ri, oi = t % 4, (t - t % 4) // 4 + 1
                for dj in range(4):
                    u = 2 * Q + dj - 1
                    rj, oj = u % 4, (u - u % 4) // 4 + 1
                    tap = c1_ref[ri, rj, :, oi:oi + 8, oj:oj + 8, :]
                    acc += jnp.dot(tap.reshape(rows, 64), w2_ref[di * 4 + dj],
                                   preferred_element_type=jnp.float32)
            y = jnp.maximum(acc + b2_ref[...], 0.0)
            c2_ref[P, Q, :, 1:9, 1:9, :] = y.reshape(nb, 8, 8, 128)
    acc3 = jnp.zeros((rows, 256), jnp.float32)
    for di in range(4):
        a, p = _TAPS[di]
        for dj in range(4):
            b_, q = _TAPS[dj]
            tap = c2_ref[p, q, :, a:a + 8, b_:b_ + 8, :]
            acc3 += jnp.dot(tap.reshape(rows, 128), w3_ref[di * 4 + dj],
                            preferred_element_type=jnp.float32)
    o_ref[...] = jnp.maximum(acc3 + b3_ref[...], 0.0)


def _encoder(xp, w1, b1, w2, b2, w3, b3, nb):
    ng = B // nb
    return pl.pallas_call(
        functools.partial(_enc_kern, nb=nb),
        grid=(ng,),
        in_specs=[
            pl.BlockSpec((nb * 1024, 16), lambda g: (g, 0)),
            pl.BlockSpec((16, 64), lambda g: (0, 0)),
            pl.BlockSpec((1, 64), lambda g: (0, 0)),
            pl.BlockSpec((16, 64, 128), lambda g: (0, 0, 0)),
            pl.BlockSpec((1, 128), lambda g: (0, 0)),
            pl.BlockSpec((16, 128, 256), lambda g: (0, 0, 0)),
            pl.BlockSpec((1, 256), lambda g: (0, 0)),
        ],
        out_specs=pl.BlockSpec((nb * 64, 256), lambda g: (g, 0)),
        out_shape=jax.ShapeDtypeStruct((B * 64, 256), jnp.float32),
        scratch_shapes=[
            pltpu.VMEM((4, 4, nb, 10, 10, 64), jnp.float32),
            pltpu.VMEM((2, 2, nb, 10, 10, 128), jnp.float32),
        ],
        compiler_params=pltpu.CompilerParams(
            dimension_semantics=("arbitrary",)),
    )(xp, w1, b1.reshape(1, 64), w2, b2.reshape(1, 128),
      w3, b3.reshape(1, 256))


_S_MID = 16
_S_T1 = 17
_S_T23 = 33
_NSTEP = 41


def _mega_kern(h3_ref, wf1_ref, bf1_ref, wfe_ref, bfe_ref, e_ref, et_ref,
               wd1_ref, bd1_ref, kt1_ref, bt1_ref, kt2_ref, bt2_ref,
               bd3_ref, bt3_ref, o_ref, acc_ref, d_ref, t1_ref):
    s = pl.program_id(0)

    @pl.when(s == 0)
    def _init():
        acc_ref[...] = jnp.zeros_like(acc_ref)

    @pl.when(s < _S_MID)
    def _fc1():
        acc_ref[...] += jnp.dot(h3_ref[...], wf1_ref[...],
                                preferred_element_type=jnp.float32)

    @pl.when(s == _S_MID)
    def _mid():
        h1 = jnp.maximum(acc_ref[...] + bf1_ref[...], 0.0)
        pe = jnp.maximum(
            jnp.dot(h1, wfe_ref[...], preferred_element_type=jnp.float32)
            + bfe_ref[...], 0.0)
        et = et_ref[...]
        e = e_ref[...]
        en2 = jnp.sum(et * et, axis=0, keepdims=True)
        cols = []
        for latent in range(LATENT):
            zl = pe[:, latent * EMB:(latent + 1) * EMB]
            dist = en2 - 2.0 * jnp.dot(zl, et,
                                       preferred_element_type=jnp.float32)
            mn = jnp.min(dist, axis=1, keepdims=True)
            iota = jax.lax.broadcasted_iota(jnp.int32, dist.shape, 1)
            idx = jnp.min(jnp.where(dist == mn, iota, KCODES), axis=1,
                          keepdims=True)
            onehot = (iota == idx).astype(jnp.float32)
            cols.append(jnp.dot(onehot, e,
                                preferred_element_type=jnp.float32))
        fcv = jnp.concatenate(cols, axis=1)
        d_ref[...] = jnp.maximum(
            jnp.dot(fcv, wd1_ref[...], preferred_element_type=jnp.float32)
            + bd1_ref[...], 0.0)

    @pl.when((s >= _S_T1) & (s < _S_T23))
    def _t1():
        j = s - _S_T1
        d = d_ref[...]
        kt = kt1_ref[...]
        for p in range(4):
            y = jnp.dot(d, kt[p], preferred_element_type=jnp.float32)
            t1_ref[pl.ds(j * 256 + p * 64, 64), :] = jnp.maximum(
                y + bt1_ref[...], 0.0)

    @pl.when(s >= _S_T23)
    def _t23():
        mred = s - _S_T23
        a2 = t1_ref[pl.ds(mred * 512, 512), :]
        y2 = jnp.maximum(
            jnp.dot(a2, kt2_ref[...], preferred_element_type=jnp.float32)
            + bt2_ref[...], 0.0)
        o_ref[...] = jax.nn.sigmoid(
            jnp.dot(y2, bd3_ref[...], preferred_element_type=jnp.float32)
            + bt3_ref[...])


def _mega(h3, Wf1, bf1, Wfe, bfe, embeds, et, Wd1, bd1, kt1r, bt1r,
          kt2r, bt2r, bd3, bt3r):
    c = lambda s: (0, 0)
    c3 = lambda s: (0, 0, 0)
    return pl.pallas_call(
        _mega_kern,
        grid=(_NSTEP,),
        in_specs=[
            pl.BlockSpec((B, 1024), lambda s: (0, jnp.minimum(s, 15))),
            pl.BlockSpec((1024, 1024), lambda s: (jnp.minimum(s, 15), 0)),
            pl.BlockSpec((1, 1024), c),
            pl.BlockSpec((1024, 2048), c),
            pl.BlockSpec((1, 2048), c),
            pl.BlockSpec((KCODES, EMB), c),
            pl.BlockSpec((EMB, KCODES), c),
            pl.BlockSpec((2048, 1024), c),
            pl.BlockSpec((1, 1024), c),
            pl.BlockSpec((4, 1024, 256),
                         lambda s: (jnp.clip(s - _S_T1, 0, 15), 0, 0)),
            pl.BlockSpec((1, 256), c),
            pl.BlockSpec((256, 2048), c),
            pl.BlockSpec((1, 2048), c),
            pl.BlockSpec((2048, 64), c),
            pl.BlockSpec((1, 64), c),
        ],
        out_specs=pl.BlockSpec((512, 64),
                               lambda s: (jnp.clip(s - _S_T23, 0, 7), 0)),
        out_shape=jax.ShapeDtypeStruct((B * 64, 64), jnp.float32),
        scratch_shapes=[
            pltpu.VMEM((B, 1024), jnp.float32),
            pltpu.VMEM((B, 1024), jnp.float32),
            pltpu.VMEM((4096, 256), jnp.float32),
        ],
        compiler_params=pltpu.CompilerParams(
            dimension_semantics=("arbitrary",)),
    )(h3, Wf1, bf1.reshape(1, 1024), Wfe, bfe.reshape(1, 2048), embeds, et,
      Wd1, bd1.reshape(1, 1024), kt1r, bt1r, kt2r, bt2r, bd3, bt3r)


def _final_transpose(y):
    return (y.reshape(8, 8, B, 4, 4, 2, 2)
             .transpose(2, 0, 3, 5, 1, 4, 6)
             .reshape(B, 64, 64, 1))


def _im2col8(x):
    x8p = jnp.pad(x.reshape(B, 8, 8, 8, 8).transpose(2, 4, 0, 1, 3),
                  ((0, 0), (0, 0), (0, 0), (1, 1), (1, 1)))
    def off(r, d):
        t = 2 * r + d - 1
        s = t % 8
        return s, (t - s) // 8 + 1
    ris = []
    for ri in range(4):
        rjs = []
        for rj in range(4):
            taps = []
            for di in range(4):
                si, oi = off(ri, di)
                for dj in range(4):
                    sj, oj = off(rj, dj)
                    taps.append(x8p[si, sj, :, oi:oi + 8, oj:oj + 8])
            rjs.append(jnp.stack(taps, axis=-1))
        ris.append(jnp.stack(rjs, axis=1))
    return jnp.stack(ris, axis=1).reshape(B * 1024, 16)


def kernel(x, Wc1, bc1, Wc2, bc2, Wc3, bc3, Wf1, bf1, Wfe, bfe, embeds,
           Wd1, bd1, Kt1, bt1, Kt2, bt2, Kt3, bt3):
    h = _encoder(_im2col8(x), Wc1.reshape(16, 64), bc1,
                 Wc2.reshape(16, 64, 128), bc2,
                 Wc3.reshape(16, 128, 256), bc3, 16)
    kt1r = Kt1.reshape(64, 1024, 256)
    bt1r = bt1.reshape(1, 256)
    kt2r = Kt2.transpose(2, 0, 1, 3).reshape(256, 2048)
    bt2r = jnp.tile(bt2, 16).reshape(1, 2048)
    bd3 = jnp.kron(jnp.eye(16, dtype=jnp.float32),
                   Kt3.transpose(2, 0, 1, 3).reshape(128, 4))
    bt3r = jnp.tile(bt3, 64).reshape(1, 64)
    y = _mega(h.reshape(B, 16384), Wf1, bf1, Wfe, bfe, embeds, embeds.T,
              Wd1, bd1, kt1r, bt1r, kt2r, bt2r, bd3, bt3r)
    return _final_transpose(y)


# speedup vs baseline: 69.4059x; 1.0123x over previous
import functools

import jax
import jax.numpy as jnp
from jax.experimental import pallas as pl
from jax.experimental.pallas import tpu as pltpu

B = 64
LATENT = 32
EMB = 64
KCODES = 512

_TAPS = [divmod(di + 1, 2) for di in range(4)]


def _enc_kern(xp_ref, w1_ref, b1_ref, w2_ref, b2_ref, w3_ref, b3_ref,
              o_ref, c1_ref, c2_ref, *, nb):
    rows = nb * 64

    @pl.when(pl.program_id(0) == 0)
    def _zero():
        c1_ref[...] = jnp.zeros_like(c1_ref)
        c2_ref[...] = jnp.zeros_like(c2_ref)

    xpr = xp_ref[...].reshape(nb, 4, 4, 8, 8, 16)
    for ri in range(4):
        for rj in range(4):
            y = jnp.maximum(
                jnp.dot(xpr[:, ri, rj].reshape(rows, 16), w1_ref[...],
                        preferred_element_type=jnp.float32) + b1_ref[...],
                0.0)
            c1_ref[ri, rj, :, 1:9, 1:9, :] = y.reshape(nb, 8, 8, 64)
    for P in range(2):
        for Q in range(2):
            acc = jnp.zeros((rows, 128), jnp.float32)
            for di in range(4):
                t = 2 * P + di - 1
                ri, oi = t % 4, (t - t % 4) // 4 + 1
                for dj in range(4):
                    u = 2 * Q + dj - 1
                    rj, oj = u % 4, (u - u % 4) // 4 + 1
                    tap = c1_ref[ri, rj, :, oi:oi + 8, oj:oj + 8, :]
                    acc += jnp.dot(tap.reshape(rows, 64), w2_ref[di * 4 + dj],
                                   preferred_element_type=jnp.float32)
            y = jnp.maximum(acc + b2_ref[...], 0.0)
            c2_ref[P, Q, :, 1:9, 1:9, :] = y.reshape(nb, 8, 8, 128)
    acc3 = jnp.zeros((rows, 256), jnp.float32)
    for di in range(4):
        a, p = _TAPS[di]
        for dj in range(4):
            b_, q = _TAPS[dj]
            tap = c2_ref[p, q, :, a:a + 8, b_:b_ + 8, :]
            acc3 += jnp.dot(tap.reshape(rows, 128), w3_ref[di * 4 + dj],
                            preferred_element_type=jnp.float32)
    o_ref[...] = jnp.maximum(acc3 + b3_ref[...], 0.0)


def _encoder(xp, w1, b1, w2, b2, w3, b3, nb):
    ng = B // nb
    return pl.pallas_call(
        functools.partial(_enc_kern, nb=nb),
        grid=(ng,),
        in_specs=[
            pl.BlockSpec((nb * 1024, 16), lambda g: (g, 0)),
            pl.BlockSpec((16, 64), lambda g: (0, 0)),
            pl.BlockSpec((1, 64), lambda g: (0, 0)),
            pl.BlockSpec((16, 64, 128), lambda g: (0, 0, 0)),
            pl.BlockSpec((1, 128), lambda g: (0, 0)),
            pl.BlockSpec((16, 128, 256), lambda g: (0, 0, 0)),
            pl.BlockSpec((1, 256), lambda g: (0, 0)),
        ],
        out_specs=pl.BlockSpec((nb * 64, 256), lambda g: (g, 0)),
        out_shape=jax.ShapeDtypeStruct((B * 64, 256), jnp.float32),
        scratch_shapes=[
            pltpu.VMEM((4, 4, nb, 10, 10, 64), jnp.float32),
            pltpu.VMEM((2, 2, nb, 10, 10, 128), jnp.float32),
        ],
        compiler_params=pltpu.CompilerParams(
            dimension_semantics=("arbitrary",)),
    )(xp, w1, b1.reshape(1, 64), w2, b2.reshape(1, 128),
      w3, b3.reshape(1, 256))


_S_MID = 8
_S_T1 = 9
_S_T23 = 25
_NSTEP = 33


def _mega_kern(h3_ref, wf1_ref, bf1_ref, wfe_ref, bfe_ref, e_ref, et_ref,
               wd1_ref, bd1_ref, kt1_ref, bt1_ref, kt2_ref, bt2_ref,
               bd3_ref, bt3_ref, o_ref, acc_ref, d_ref, t1_ref):
    s = pl.program_id(0)

    @pl.when(s == 0)
    def _init():
        acc_ref[...] = jnp.zeros_like(acc_ref)

    @pl.when(s < _S_MID)
    def _fc1():
        acc_ref[...] += jnp.dot(h3_ref[...], wf1_ref[...],
                                preferred_element_type=jnp.float32)

    @pl.when(s == _S_MID)
    def _mid():
        h1 = jnp.maximum(acc_ref[...] + bf1_ref[...], 0.0)
        pe = jnp.maximum(
            jnp.dot(h1, wfe_ref[...], preferred_element_type=jnp.float32)
            + bfe_ref[...], 0.0)
        et = et_ref[...]
        e = e_ref[...]
        en2 = 0.25 * jnp.sum(et * et, axis=0, keepdims=True)
        cols = []
        for latent in range(LATENT):
            zl = pe[:, latent * EMB:(latent + 1) * EMB]
            dist = en2 + jnp.dot(zl, et,
                                 preferred_element_type=jnp.float32)
            mn = jnp.min(dist, axis=1, keepdims=True)
            iota = jax.lax.broadcasted_iota(jnp.int32, dist.shape, 1)
            idx = jnp.min(jnp.where(dist == mn, iota, KCODES), axis=1,
                          keepdims=True)
            onehot = (iota == idx).astype(jnp.float32)
            cols.append(jnp.dot(onehot, e,
                                preferred_element_type=jnp.float32))
        fcv = jnp.concatenate(cols, axis=1)
        d_ref[...] = jnp.maximum(
            jnp.dot(fcv, wd1_ref[...], preferred_element_type=jnp.float32)
            + bd1_ref[...], 0.0)

    @pl.when((s >= _S_T1) & (s < _S_T23))
    def _t1():
        j = s - _S_T1
        d = d_ref[...]
        kt = kt1_ref[...]
        for p in range(4):
            y = jnp.dot(d, kt[p], preferred_element_type=jnp.float32)
            t1_ref[pl.ds(j * 256 + p * 64, 64), :] = jnp.maximum(
                y + bt1_ref[...], 0.0)

    @pl.when(s >= _S_T23)
    def _t23():
        mred = s - _S_T23
        a2 = t1_ref[pl.ds(mred * 512, 512), :]
        y2 = jnp.maximum(
            jnp.dot(a2, kt2_ref[...], preferred_element_type=jnp.float32)
            + bt2_ref[...], 0.0)
        o_ref[...] = jax.nn.sigmoid(
            jnp.dot(y2, bd3_ref[...], preferred_element_type=jnp.float32)
            + bt3_ref[...])


def _mega(h3, Wf1, bf1, Wfe, bfe, embeds, et, Wd1, bd1, kt1r, bt1r,
          kt2r, bt2r, bd3, bt3r):
    c = lambda s: (0, 0)
    c3 = lambda s: (0, 0, 0)
    return pl.pallas_call(
        _mega_kern,
        grid=(_NSTEP,),
        in_specs=[
            pl.BlockSpec((B, 2048), lambda s: (0, jnp.minimum(s, 7))),
            pl.BlockSpec((2048, 1024), lambda s: (jnp.minimum(s, 7), 0)),
            pl.BlockSpec((1, 1024), c),
            pl.BlockSpec((1024, 2048), c),
            pl.BlockSpec((1, 2048), c),
            pl.BlockSpec((KCODES, EMB), c),
            pl.BlockSpec((EMB, KCODES), c),
            pl.BlockSpec((2048, 1024), c),
            pl.BlockSpec((1, 1024), c),
            pl.BlockSpec((4, 1024, 256),
                         lambda s: (jnp.clip(s - _S_T1, 0, 15), 0, 0)),
            pl.BlockSpec((1, 256), c),
            pl.BlockSpec((256, 2048), c),
            pl.BlockSpec((1, 2048), c),
            pl.BlockSpec((2048, 64), c),
            pl.BlockSpec((1, 64), c),
        ],
        out_specs=pl.BlockSpec((512, 64),
                               lambda s: (jnp.clip(s - _S_T23, 0, 7), 0)),
        out_shape=jax.ShapeDtypeStruct((B * 64, 64), jnp.float32),
        scratch_shapes=[
            pltpu.VMEM((B, 1024), jnp.float32),
            pltpu.VMEM((B, 1024), jnp.float32),
            pltpu.VMEM((4096, 256), jnp.float32),
        ],
        compiler_params=pltpu.CompilerParams(
            dimension_semantics=("arbitrary",)),
    )(h3, Wf1, bf1.reshape(1, 1024), Wfe, bfe.reshape(1, 2048), embeds, et,
      Wd1, bd1.reshape(1, 1024), kt1r, bt1r, kt2r, bt2r, bd3, bt3r)


def _final_transpose(y):
    return (y.reshape(8, 8, B, 4, 4, 2, 2)
             .transpose(2, 0, 3, 5, 1, 4, 6)
             .reshape(B, 64, 64, 1))


def _im2col8(x):
    x8p = jnp.pad(x.reshape(B, 8, 8, 8, 8).transpose(2, 4, 0, 1, 3),
                  ((0, 0), (0, 0), (0, 0), (1, 1), (1, 1)))
    def off(r, d):
        t = 2 * r + d - 1
        s = t % 8
        return s, (t - s) // 8 + 1
    ris = []
    for ri in range(4):
        rjs = []
        for rj in range(4):
            taps = []
            for di in range(4):
                si, oi = off(ri, di)
                for dj in range(4):
                    sj, oj = off(rj, dj)
                    taps.append(x8p[si, sj, :, oi:oi + 8, oj:oj + 8])
            rjs.append(jnp.stack(taps, axis=-1))
        ris.append(jnp.stack(rjs, axis=1))
    return jnp.stack(ris, axis=1).reshape(B * 1024, 16)


def kernel(x, Wc1, bc1, Wc2, bc2, Wc3, bc3, Wf1, bf1, Wfe, bfe, embeds,
           Wd1, bd1, Kt1, bt1, Kt2, bt2, Kt3, bt3):
    h = _encoder(_im2col8(x), Wc1.reshape(16, 64), bc1,
                 Wc2.reshape(16, 64, 128), bc2,
                 Wc3.reshape(16, 128, 256), bc3, 16)
    kt1r = Kt1.reshape(64, 1024, 256)
    bt1r = bt1.reshape(1, 256)
    kt2r = Kt2.transpose(2, 0, 1, 3).reshape(256, 2048)
    bt2r = jnp.tile(bt2, 16).reshape(1, 2048)
    bd3 = jnp.kron(jnp.eye(16, dtype=jnp.float32),
                   Kt3.transpose(2, 0, 1, 3).reshape(128, 4))
    bt3r = jnp.tile(bt3, 64).reshape(1, 64)
    y = _mega(h.reshape(B, 16384), Wf1, bf1, Wfe, bfe, embeds,
              -2.0 * embeds.T,
              Wd1, bd1, kt1r, bt1r, kt2r, bt2r, bd3, bt3r)
    return _final_transpose(y)
